# Initial kernel scaffold; baseline (speedup 1.0000x reference)
import jax
import jax.numpy as jnp
from jax.experimental import pallas as pl


def kernel(x, edge_index, edge_type, norm, prev_h, emb_rel, weight_neighbor, loop_weight, skip_connect_weight, skip_connect_bias):
    raise NotImplementedError("write your pallas kernel here")


# speedup vs baseline: 5.2401x; 5.2401x over previous
import functools

import jax
import jax.numpy as jnp
from jax import lax
from jax.experimental import pallas as pl
from jax.experimental.pallas import tpu as pltpu
from jax.experimental.pallas import tpu_sc as plsc

N = 10000
E = 320000
D = 128
R = 200

NC = 2
NS = 16
NW = NC * NS
EDGES_PER_TILE = E // NW
NPAD = 10240
ROWS_PER_TILE = NPAD // NS
K = 80
CHUNKS = EDGES_PER_TILE // K

BLK = 2000


def _dense_body(x_ref, ph_ref, wn_ref, wl_ref, ws_ref, b_ref,
                xw_ref, loop_ref, gate_ref):
    xb = x_ref[...]
    xw_ref[...] = jnp.dot(xb, wn_ref[...], preferred_element_type=jnp.float32)
    loop_ref[...] = jnp.dot(xb, wl_ref[...], preferred_element_type=jnp.float32)
    z = jnp.dot(ph_ref[...], ws_ref[...], preferred_element_type=jnp.float32)
    gate_ref[...] = jax.nn.sigmoid(z + b_ref[...])


def _tc_dense(x, prev_h, wn, wl, ws, b2d):
    grid = (N // BLK,)
    row_spec = pl.BlockSpec((BLK, D), lambda i: (i, 0))
    w_spec = pl.BlockSpec((D, D), lambda i: (0, 0))
    b_spec = pl.BlockSpec((1, D), lambda i: (0, 0))
    out_sds = jax.ShapeDtypeStruct((N, D), jnp.float32)
    return pl.pallas_call(
        _dense_body,
        grid=grid,
        in_specs=[row_spec, row_spec, w_spec, w_spec, w_spec, b_spec],
        out_specs=[row_spec, row_spec, row_spec],
        out_shape=[out_sds, out_sds, out_sds],
    )(x, prev_h, wn, wl, ws, b2d)


def _embw_body(e_ref, w_ref, o_ref):
    o_ref[...] = jnp.dot(e_ref[...], w_ref[...],
                         preferred_element_type=jnp.float32)


def _tc_embw(emb_rel, wn):
    return pl.pallas_call(
        _embw_body,
        out_shape=jax.ShapeDtypeStruct((R, D), jnp.float32),
    )(emb_rel, wn)


def _sc_edge_body(xw_hbm, embw_hbm, src_hbm, dst_hbm, typ_hbm, zeros_hbm,
                  out_hbm, acc, src_v, dst_v, typ_v, xrow_v, rrow_v,
                  sem1, sem2):
    c = lax.axis_index("c")
    s = lax.axis_index("s")
    pltpu.sync_copy(zeros_hbm, acc.at[pl.ds(s * ROWS_PER_TILE, ROWS_PER_TILE)])
    plsc.subcore_barrier()

    tile_base = (c * NS + s) * EDGES_PER_TILE

    def chunk(i, carry):
        base = tile_base + i * K
        pltpu.sync_copy(src_hbm.at[pl.ds(base, K)], src_v)
        pltpu.sync_copy(typ_hbm.at[pl.ds(base, K)], typ_v)
        pltpu.sync_copy(dst_hbm.at[pl.ds(base, K)], dst_v)
        cp1 = pltpu.async_copy(xw_hbm.at[src_v], xrow_v, sem1)
        cp2 = pltpu.async_copy(embw_hbm.at[typ_v], rrow_v, sem2)
        cp1.wait()
        cp2.wait()
        pltpu.sync_copy(xrow_v, acc.at[dst_v], add=True)
        pltpu.sync_copy(rrow_v, acc.at[dst_v], add=True)
        return carry

    lax.fori_loop(0, CHUNKS, chunk, 0)
    plsc.subcore_barrier()
    pltpu.sync_copy(acc.at[pl.ds(s * ROWS_PER_TILE, ROWS_PER_TILE)],
                    out_hbm.at[c, pl.ds(s * ROWS_PER_TILE, ROWS_PER_TILE)])


def _sc_edges(xw, embw, src, dst, etype, zeros):
    mesh = plsc.VectorSubcoreMesh(core_axis_name="c", subcore_axis_name="s")
    fn = functools.partial(
        pl.kernel,
        mesh=mesh,
        out_type=jax.ShapeDtypeStruct((NC, NPAD, D), jnp.float32),
        scratch_types=[
            pltpu.VMEM_SHARED((NPAD, D), jnp.float32),
            pltpu.VMEM((K,), jnp.int32),
            pltpu.VMEM((K,), jnp.int32),
            pltpu.VMEM((K,), jnp.int32),
            pltpu.VMEM((K, D), jnp.float32),
            pltpu.VMEM((K, D), jnp.float32),
            pltpu.SemaphoreType.DMA,
            pltpu.SemaphoreType.DMA,
        ],
    )(_sc_edge_body)
    return fn(xw, embw, src, dst, etype, zeros)


def _final_body(p_ref, norm_ref, loop_ref, gate_ref, prev_ref, o_ref):
    agg = p_ref[0] + p_ref[1]
    h = agg * norm_ref[...] + loop_ref[...]
    g = gate_ref[...]
    o_ref[...] = g * h + (1.0 - g) * prev_ref[...]


def _tc_final(partials, norm, loop_m, gate, prev_h):
    grid = (N // BLK,)
    row_spec = pl.BlockSpec((BLK, D), lambda i: (i, 0))
    p_spec = pl.BlockSpec((NC, BLK, D), lambda i: (0, i, 0))
    n_spec = pl.BlockSpec((BLK, 1), lambda i: (i, 0))
    return pl.pallas_call(
        _final_body,
        grid=grid,
        in_specs=[p_spec, n_spec, row_spec, row_spec, row_spec],
        out_specs=pl.BlockSpec((BLK, D), lambda i: (i, 0)),
        out_shape=jax.ShapeDtypeStruct((N, D), jnp.float32),
    )(partials, norm, loop_m, gate, prev_h)


def kernel(x, edge_index, edge_type, norm, prev_h, emb_rel,
           weight_neighbor, loop_weight, skip_connect_weight,
           skip_connect_bias):
    src = edge_index[0]
    dst = edge_index[1]
    b2d = skip_connect_bias.reshape(1, D)
    xw, loop_m, gate = _tc_dense(x, prev_h, weight_neighbor, loop_weight,
                                 skip_connect_weight, b2d)
    embw = _tc_embw(emb_rel, weight_neighbor)
    zeros = jnp.zeros((ROWS_PER_TILE, D), jnp.float32)
    partials = _sc_edges(xw, embw, src, dst, edge_type, zeros)
    return _tc_final(partials[:, :N], norm, loop_m, gate, prev_h)
